# Initial kernel scaffold; baseline (speedup 1.0000x reference)
import jax
import jax.numpy as jnp
from jax.experimental import pallas as pl


def kernel(flows, W1, b1, W2, b2, W3, b3):
    raise NotImplementedError("write your pallas kernel here")


# speedup vs baseline: 2886.3073x; 2886.3073x over previous
import jax
import jax.numpy as jnp
from jax.experimental import pallas as pl

B, N = 4, 512
RNN, INTER = 128, 256


def _encoder_kernel(f_ref, w1_ref, b1_ref, w2_ref, b2_ref, w3_ref, b3_ref,
                    out_ref):
    f = f_ref[...]

    deg = jnp.sum(f, axis=0) + 1.0
    dinv = jnp.where(deg > 0.0, jax.lax.rsqrt(deg), 0.0)
    dcol = dinv[:, None]

    def layer(x, w, b):
        h = jax.lax.dot_general(
            x, w, (((1,), (0,)), ((), ())),
            preferred_element_type=jnp.float32)
        y = h * dcol
        z = jax.lax.dot_general(
            f, y, (((0,), (0,)), ((), ())),
            preferred_element_type=jnp.float32)
        return (z + y) * dcol + b[None, :]

    x = layer(f, w1_ref[...], b1_ref[...])
    x = layer(x, w2_ref[...], b2_ref[...])
    x = layer(x, w3_ref[...], b3_ref[...])
    out_ref[...] = x


def kernel(flows, W1, b1, W2, b2, W3, b3):
    full = lambda shape: pl.BlockSpec(shape, lambda b: (0,) * len(shape))
    return pl.pallas_call(
        _encoder_kernel,
        grid=(B,),
        in_specs=[
            pl.BlockSpec((None, N, N), lambda b: (b, 0, 0)),
            full((N, RNN)),
            full((RNN,)),
            full((RNN, INTER)),
            full((INTER,)),
            full((INTER, RNN)),
            full((RNN,)),
        ],
        out_specs=pl.BlockSpec((None, N, RNN), lambda b: (b, 0, 0)),
        out_shape=jax.ShapeDtypeStruct((B, N, RNN), jnp.float32),
    )(flows, W1, b1, W2, b2, W3, b3)
